# baseline (speedup 1.0000x reference)
import functools

import jax
import jax.numpy as jnp
from jax import lax
from jax.experimental import pallas as pl
from jax.experimental.pallas import tpu as pltpu
from jax.experimental.pallas import tpu_sc as plsc

_B = 16384
_K = 64
_N = 1000000

_info = plsc.get_sparse_core_info()
_NC = _info.num_cores
_NS = _info.num_subcores
_NW = _NC * _NS
_BPW = _B // _NW
_R = 8

_mesh = plsc.VectorSubcoreMesh(core_axis_name="c", subcore_axis_name="s")


@functools.partial(
    pl.kernel,
    mesh=_mesh,
    compiler_params=pltpu.CompilerParams(use_tc_tiling_on_sc=False),
    out_type=[
        jax.ShapeDtypeStruct((_K, _B), jnp.float32),
        jax.ShapeDtypeStruct((_K, _B), jnp.float32),
    ],
    scratch_types=[
        pltpu.VMEM((_BPW,), jnp.int32),
        pltpu.VMEM((_BPW,), jnp.int32),
        pltpu.VMEM((_K, _BPW), jnp.float32),
        pltpu.VMEM((_K, _BPW), jnp.float32),
        pltpu.SemaphoreType.DMA,
        pltpu.SemaphoreType.DMA,
    ],
)
def _gather_t(
    gu_t,
    gi_t,
    users_hbm,
    items_hbm,
    out_u,
    out_i,
    uidx_v,
    iidx_v,
    urows_v,
    irows_v,
    sem_u,
    sem_i,
):
    wid = lax.axis_index("s") * _NC + lax.axis_index("c")
    base = wid * _BPW
    pltpu.sync_copy(users_hbm.at[pl.ds(base, _BPW)], uidx_v)
    pltpu.sync_copy(items_hbm.at[pl.ds(base, _BPW)], iidx_v)

    def fire(k):
        pltpu.async_copy(gu_t.at[k].at[uidx_v], urows_v.at[k], sem_u)
        pltpu.async_copy(gi_t.at[k].at[iidx_v], irows_v.at[k], sem_i)

    def drain(kk):
        pltpu.make_async_copy(gu_t.at[kk].at[uidx_v], urows_v.at[kk], sem_u).wait()
        pltpu.make_async_copy(gi_t.at[kk].at[iidx_v], irows_v.at[kk], sem_i).wait()

    def body(k, _):
        fire(k)

        @pl.when(k >= _R)
        def _():
            drain(k - _R)

        return 0

    lax.fori_loop(0, _K, body, 0)

    def tail(t, _):
        drain(_K - _R + t)
        return 0

    lax.fori_loop(0, _R, tail, 0)

    pltpu.sync_copy(urows_v, out_u.at[:, pl.ds(base, _BPW)])
    pltpu.sync_copy(irows_v, out_i.at[:, pl.ds(base, _BPW)])


def kernel(Gu, Gi, users, items):
    out_tu, out_ti = _gather_t(
        Gu.T, Gi.T, users.astype(jnp.int32), items.astype(jnp.int32)
    )
    return (out_tu.T, out_ti.T)


# speedup vs baseline: 8.5234x; 8.5234x over previous
import functools

import jax
import jax.numpy as jnp
from jax import lax
from jax.experimental import pallas as pl
from jax.experimental.pallas import tpu as pltpu
from jax.experimental.pallas import tpu_sc as plsc

_B = 16384
_K = 64
_N = 1000000

_info = plsc.get_sparse_core_info()
_NC = _info.num_cores
_NS = _info.num_subcores
_NW = _NC * _NS
_BPW = _B // _NW
_CHUNKS = _BPW // 16

_mesh = plsc.VectorSubcoreMesh(core_axis_name="c", subcore_axis_name="s")


@functools.partial(
    pl.kernel,
    mesh=_mesh,
    compiler_params=pltpu.CompilerParams(needs_layout_passes=False),
    out_type=[
        jax.ShapeDtypeStruct((_K, _B), jnp.float32),
        jax.ShapeDtypeStruct((_K, _B), jnp.float32),
    ],
    scratch_types=[
        pltpu.VMEM((_BPW,), jnp.int32),
        pltpu.VMEM((_BPW,), jnp.int32),
        pltpu.VMEM((_BPW,), jnp.int32),
        pltpu.VMEM((_BPW, 2 * _K), jnp.float32),
        pltpu.VMEM((_K, _BPW), jnp.float32),
        pltpu.SemaphoreType.DMA,
    ],
)
def _gather_pairs(
    gu2,
    gi2,
    users_hbm,
    items_hbm,
    out_u,
    out_i,
    idx_v,
    half_v,
    pair_v,
    rows_v,
    outt_v,
    sem,
):
    wid = lax.axis_index("s") * _NC + lax.axis_index("c")
    base = wid * _BPW

    def run_table(table, src_hbm, out_hbm):
        pltpu.sync_copy(src_hbm.at[pl.ds(base, _BPW)], idx_v)

        def pb(c, _):
            r = idx_v[pl.ds(c * 16, 16)]
            half_v[pl.ds(c * 16, 16)] = (r & 1) << 6
            pair_v[pl.ds(c * 16, 16)] = r >> 1
            return 0

        lax.fori_loop(0, _CHUNKS, pb, 0)

        pltpu.async_copy(table.at[pair_v], rows_v, sem).wait()

        def kb(k, _):
            def cb(c, _):
                j = c * 16 + lax.iota(jnp.int32, 16)
                col = half_v[pl.ds(c * 16, 16)] + k
                outt_v[k, pl.ds(c * 16, 16)] = plsc.load_gather(rows_v, [j, col])
                return 0

            lax.fori_loop(0, _CHUNKS, cb, 0)
            return 0

        lax.fori_loop(0, _K, kb, 0)

        pltpu.sync_copy(outt_v, out_hbm.at[:, pl.ds(base, _BPW)])

    run_table(gu2, users_hbm, out_u)
    run_table(gi2, items_hbm, out_i)


def kernel(Gu, Gi, users, items):
    gu2 = Gu.reshape(_N // 2, 2 * _K)
    gi2 = Gi.reshape(_N // 2, 2 * _K)
    out_tu, out_ti = _gather_pairs(
        gu2, gi2, users.astype(jnp.int32), items.astype(jnp.int32)
    )
    return (out_tu.T, out_ti.T)


# speedup vs baseline: 8.8561x; 1.0390x over previous
import functools

import jax
import jax.numpy as jnp
from jax import lax
from jax.experimental import pallas as pl
from jax.experimental.pallas import tpu as pltpu
from jax.experimental.pallas import tpu_sc as plsc

_B = 16384
_K = 64
_N = 1000000

_info = plsc.get_sparse_core_info()
_NC = _info.num_cores
_NS = _info.num_subcores
_NW = _NC * _NS
_BPW = _B // _NW
_CHUNKS = _BPW // 16

_mesh = plsc.VectorSubcoreMesh(core_axis_name="c", subcore_axis_name="s")


@functools.partial(
    pl.kernel,
    mesh=_mesh,
    compiler_params=pltpu.CompilerParams(needs_layout_passes=False),
    out_type=jax.ShapeDtypeStruct((_K, _B), jnp.float32),
    scratch_types=[
        pltpu.VMEM((_BPW,), jnp.int32),
        pltpu.VMEM((_BPW,), jnp.int32),
        pltpu.VMEM((_BPW,), jnp.int32),
        pltpu.VMEM((_BPW, 2 * _K), jnp.float32),
        pltpu.VMEM((_K, _BPW), jnp.float32),
        pltpu.SemaphoreType.DMA,
    ],
)
def _gather_one(
    table2,
    src_hbm,
    out_t,
    idx_v,
    half_v,
    pair_v,
    rows_v,
    outt_v,
    sem,
):
    wid = lax.axis_index("s") * _NC + lax.axis_index("c")
    base = wid * _BPW
    pltpu.sync_copy(src_hbm.at[pl.ds(base, _BPW)], idx_v)

    def pb(c, _):
        r = idx_v[pl.ds(c * 16, 16)]
        half_v[pl.ds(c * 16, 16)] = (r & 1) << 6
        pair_v[pl.ds(c * 16, 16)] = r >> 1
        return 0

    lax.fori_loop(0, _CHUNKS, pb, 0)

    pltpu.async_copy(table2.at[pair_v], rows_v, sem).wait()

    def kb(k, _):
        for c in range(_CHUNKS):
            j = c * 16 + lax.iota(jnp.int32, 16)
            col = half_v[pl.ds(c * 16, 16)] + k
            outt_v[k, pl.ds(c * 16, 16)] = plsc.load_gather(rows_v, [j, col])
        return 0

    lax.fori_loop(0, _K, kb, 0)

    pltpu.sync_copy(outt_v, out_t.at[:, pl.ds(base, _BPW)])


def kernel(Gu, Gi, users, items):
    gu2 = Gu.reshape(_N // 2, 2 * _K)
    gi2 = Gi.reshape(_N // 2, 2 * _K)
    out_tu = _gather_one(gu2, users.astype(jnp.int32))
    out_ti = _gather_one(gi2, items.astype(jnp.int32))
    return (out_tu.T, out_ti.T)
